# baseline (speedup 1.0000x reference)
import jax
import jax.numpy as jnp
from jax.experimental import pallas as pl
from jax.experimental.pallas import tpu as pltpu

TOP_K = 2
IMPORTANCE_REG = 0.01

_TT = 512
_TH = 1024


def _gate_kernel(hm_ref, gw_ref, gb_ref, code_ref,
                 combine_ref, probs_part_ref, comb_part_ref, code_bf_ref):
    logits = jnp.dot(hm_ref[...], gw_ref[...],
                     preferred_element_type=jnp.float32) + gb_ref[...][None, :]
    lane = jax.lax.broadcasted_iota(jnp.int32, logits.shape, 1)
    v1 = jnp.max(logits, axis=1, keepdims=True)
    i1 = jnp.argmax(logits, axis=1)[:, None]
    masked = jnp.where(lane == i1, -jnp.inf, logits)
    v2 = jnp.max(masked, axis=1, keepdims=True)
    i2 = jnp.argmax(masked, axis=1)[:, None]
    e2 = jnp.exp(v2 - v1)
    g1 = 1.0 / (1.0 + e2)
    g2 = e2 * g1
    combine = jnp.where(lane == i1, g1, 0.0) + jnp.where(lane == i2, g2, 0.0)
    combine_ref[...] = combine

    ex = jnp.exp(logits - v1)
    probs = ex / jnp.sum(ex, axis=1, keepdims=True)
    probs_part_ref[0, 0, :] = jnp.sum(probs, axis=0)
    comb_part_ref[0, 0, :] = jnp.sum(combine, axis=0)

    code_bf_ref[...] = code_ref[...].astype(jnp.bfloat16)


def _moe_kernel(temp_ref, fi_ref, code_bf_ref, comb_ref,
                w1_ref, w2_ref,
                cmw_ref, fw1_ref, fw2_ref, Fw1_ref, Fw2_ref,
                imp_ref, out_ref, cmw_bf_ref, fw1_bf_ref, fw2_bf_ref,
                Fw1_bf_ref, Fw2_bf_ref):
    bf = jnp.bfloat16
    e = pl.program_id(0)

    t = jnp.clip(temp_ref[0, 0], 0.1, 5.0)
    fi = fi_ref[0] / t
    fm = jnp.max(fi, axis=1, keepdims=True)
    fe = jnp.exp(fi - fm)
    imp = fe / jnp.sum(fe, axis=1, keepdims=True)
    imp_ref[0] = imp
    w1s = (w1_ref[0] * imp[0][:, None]).astype(bf)

    x = code_bf_ref[...]
    h1 = jnp.maximum(
        jnp.dot(x, w1s, preferred_element_type=jnp.float32),
        0.0).astype(bf)
    y = jnp.dot(h1, w2_ref[0].astype(bf),
                preferred_element_type=jnp.float32)
    comb = comb_ref[...]
    lane = jax.lax.broadcasted_iota(jnp.int32, comb.shape, 1)
    col = jnp.sum(jnp.where(lane == e, comb, 0.0), axis=1, keepdims=True)

    @pl.when(e == 0)
    def _():
        out_ref[...] = col * y

    @pl.when(e != 0)
    def _():
        out_ref[...] += col * y

    cmw_bf_ref[...] = cmw_ref[...].astype(bf)
    fw1_bf_ref[...] = fw1_ref[...].astype(bf)
    fw2_bf_ref[...] = fw2_ref[...].astype(bf)
    Fw1_bf_ref[...] = Fw1_ref[...].astype(bf)
    Fw2_bf_ref[...] = Fw2_ref[...].astype(bf)


def _softplus(x):
    return jnp.maximum(x, 0.0) + jnp.log1p(jnp.exp(-jnp.abs(x)))


def _head_kernel(moe_ref, code_bf_ref, u_ref, cmwc_ref, cmwu_ref,
                 fw1_ref, fw2_ref,
                 Fw1_ref, Fw2_ref,
                 probs_part_ref, comb_part_ref, imp_ref,
                 y_ref, losses_ref, spars_acc):
    bf = jnp.bfloat16
    i = pl.program_id(0)
    n = pl.num_programs(0)
    ub = u_ref[...].astype(bf)
    mask = jax.nn.sigmoid(
        jnp.dot(code_bf_ref[...], cmwc_ref[...],
                preferred_element_type=jnp.float32)
        + jnp.dot(ub, cmwu_ref[...], preferred_element_type=jnp.float32))
    f1 = jnp.maximum(
        jnp.dot(ub, fw1_ref[...], preferred_element_type=jnp.float32), 0.0)
    f_u = jnp.dot(f1.astype(bf), fw2_ref[...],
                  preferred_element_type=jnp.float32)
    out = moe_ref[...] * (1.0 - mask) + f_u * mask
    h = jnp.dot(out.astype(bf), Fw1_ref[...],
                preferred_element_type=jnp.float32)
    h = h * jnp.tanh(_softplus(h))
    y_ref[...] = jnp.tanh(
        jnp.dot(h.astype(bf), Fw2_ref[...],
                preferred_element_type=jnp.float32))

    part = jnp.broadcast_to(jnp.sum(jnp.abs(mask)), spars_acc.shape)

    @pl.when(i == 0)
    def _():
        spars_acc[...] = part

    @pl.when(i != 0)
    def _():
        spars_acc[...] += part

    @pl.when(i == n - 1)
    def _():
        T = moe_ref.shape[0] * n
        H = moe_ref.shape[1]
        E = probs_part_ref.shape[2]
        probs_mean = jnp.sum(probs_part_ref[...], axis=(0, 1)) / T
        load = jnp.sum(comb_part_ref[...], axis=(0, 1)) / T
        lb_loss = E * jnp.sum(probs_mean * load)
        impv = imp_ref[...]
        imp_loss = IMPORTANCE_REG * jnp.mean(jnp.sum(impv * impv, axis=-1))
        aux = lb_loss + imp_loss
        spars = jnp.max(spars_acc[...]) / (T * H)
        lane1 = jax.lax.broadcasted_iota(jnp.int32, losses_ref.shape, 1)
        v = jnp.where(lane1 == 0, aux + spars,
                      jnp.where(lane1 == 1, aux, spars))
        losses_ref[...] = v


def kernel(h_modulated, code_emb, u, gate_w, gate_b, feature_importance,
           importance_temperature, expert_w1, expert_b1, expert_w2, expert_b2,
           conf_mask_w, conf_mask_b, f_conf_w1, f_conf_b1, f_conf_w2,
           f_conf_b2, final_w1, final_b1, final_w2, final_b2):
    T, D = h_modulated.shape
    C = code_emb.shape[1]
    U = u.shape[1]
    E = gate_w.shape[1]
    H = expert_b1.shape[1]
    G = T // _TT
    bf = jnp.bfloat16

    temp = importance_temperature.reshape(1, 1)
    fi3 = feature_importance.reshape(E, 1, C)

    combine, probs_part, comb_part, code_bf = pl.pallas_call(
        _gate_kernel,
        grid=(G,),
        in_specs=[
            pl.BlockSpec((_TT, D), lambda i: (i, 0)),
            pl.BlockSpec((D, E), lambda i: (0, 0)),
            pl.BlockSpec((E,), lambda i: (0,)),
            pl.BlockSpec((_TT, C), lambda i: (i, 0)),
        ],
        out_specs=[
            pl.BlockSpec((_TT, E), lambda i: (i, 0)),
            pl.BlockSpec((1, 1, E), lambda i: (i, 0, 0)),
            pl.BlockSpec((1, 1, E), lambda i: (i, 0, 0)),
            pl.BlockSpec((_TT, C), lambda i: (i, 0)),
        ],
        out_shape=[
            jax.ShapeDtypeStruct((T, E), jnp.float32),
            jax.ShapeDtypeStruct((G, 1, E), jnp.float32),
            jax.ShapeDtypeStruct((G, 1, E), jnp.float32),
            jax.ShapeDtypeStruct((T, C), bf),
        ],
    )(h_modulated, gate_w, gate_b, code_emb)

    CMR = (C + U) // E
    FW1R = U // E
    FW2R = H // E
    FR1 = H // E
    FR2 = 2 * H // E

    (imp, moe_out, cmw_bf, fw1_bf, fw2_bf, Fw1_bf, Fw2_bf) = pl.pallas_call(
        _moe_kernel,
        grid=(E,),
        in_specs=[
            pl.BlockSpec(memory_space=pltpu.SMEM),
            pl.BlockSpec((1, 1, C), lambda e: (e, 0, 0)),
            pl.BlockSpec((T, C), lambda e: (0, 0)),
            pl.BlockSpec((T, E), lambda e: (0, 0)),
            pl.BlockSpec((1, C, H), lambda e: (e, 0, 0)),
            pl.BlockSpec((1, H, H), lambda e: (e, 0, 0)),
            pl.BlockSpec((CMR, H), lambda e: (e, 0)),
            pl.BlockSpec((FW1R, H), lambda e: (e, 0)),
            pl.BlockSpec((FW2R, H), lambda e: (e, 0)),
            pl.BlockSpec((FR1, 2 * H), lambda e: (e, 0)),
            pl.BlockSpec((FR2, H), lambda e: (e, 0)),
        ],
        out_specs=[
            pl.BlockSpec((1, 1, C), lambda e: (e, 0, 0)),
            pl.BlockSpec((T, H), lambda e: (0, 0)),
            pl.BlockSpec((CMR, H), lambda e: (e, 0)),
            pl.BlockSpec((FW1R, H), lambda e: (e, 0)),
            pl.BlockSpec((FW2R, H), lambda e: (e, 0)),
            pl.BlockSpec((FR1, 2 * H), lambda e: (e, 0)),
            pl.BlockSpec((FR2, H), lambda e: (e, 0)),
        ],
        out_shape=[
            jax.ShapeDtypeStruct((E, 1, C), jnp.float32),
            jax.ShapeDtypeStruct((T, H), jnp.float32),
            jax.ShapeDtypeStruct((C + U, H), bf),
            jax.ShapeDtypeStruct((U, H), bf),
            jax.ShapeDtypeStruct((H, H), bf),
            jax.ShapeDtypeStruct((H, 2 * H), bf),
            jax.ShapeDtypeStruct((2 * H, H), bf),
        ],
    )(temp, fi3, code_bf, combine, expert_w1, expert_w2,
      conf_mask_w, f_conf_w1, f_conf_w2, final_w1, final_w2)

    y, losses = pl.pallas_call(
        _head_kernel,
        grid=(T // _TH,),
        in_specs=[
            pl.BlockSpec((_TH, H), lambda i: (i, 0)),
            pl.BlockSpec((_TH, C), lambda i: (i, 0)),
            pl.BlockSpec((_TH, U), lambda i: (i, 0)),
            pl.BlockSpec((C, H), lambda i: (0, 0)),
            pl.BlockSpec((U, H), lambda i: (C // U, 0)),
            pl.BlockSpec((U, H), lambda i: (0, 0)),
            pl.BlockSpec((H, H), lambda i: (0, 0)),
            pl.BlockSpec((H, 2 * H), lambda i: (0, 0)),
            pl.BlockSpec((2 * H, H), lambda i: (0, 0)),
            pl.BlockSpec((G, 1, E), lambda i: (0, 0, 0)),
            pl.BlockSpec((G, 1, E), lambda i: (0, 0, 0)),
            pl.BlockSpec((E, 1, C), lambda i: (0, 0, 0)),
        ],
        out_specs=[
            pl.BlockSpec((_TH, H), lambda i: (i, 0)),
            pl.BlockSpec((1, 128), lambda i: (0, 0)),
        ],
        out_shape=[
            jax.ShapeDtypeStruct((T, H), jnp.float32),
            jax.ShapeDtypeStruct((1, 128), jnp.float32),
        ],
        scratch_shapes=[pltpu.VMEM((1, 128), jnp.float32)],
    )(moe_out, code_bf, u, cmw_bf, cmw_bf,
      fw1_bf, fw2_bf,
      Fw1_bf, Fw2_bf,
      probs_part, comb_part, imp)

    total_loss = losses[0, 0]
    aux_loss = losses[0, 1]
    sparsity_loss = losses[0, 2]
    return (y, total_loss, aux_loss, sparsity_loss)


# speedup vs baseline: 1.0215x; 1.0215x over previous
import jax
import jax.numpy as jnp
from jax.experimental import pallas as pl
from jax.experimental.pallas import tpu as pltpu

TOP_K = 2
IMPORTANCE_REG = 0.01

_TT = 256
_TH = 512


def _gate_kernel(hm_ref, gw_ref, gb_ref, code_ref,
                 combine_ref, probs_part_ref, comb_part_ref, code_bf_ref):
    logits = jnp.dot(hm_ref[...], gw_ref[...],
                     preferred_element_type=jnp.float32) + gb_ref[...][None, :]
    lane = jax.lax.broadcasted_iota(jnp.int32, logits.shape, 1)
    v1 = jnp.max(logits, axis=1, keepdims=True)
    i1 = jnp.argmax(logits, axis=1)[:, None]
    masked = jnp.where(lane == i1, -jnp.inf, logits)
    v2 = jnp.max(masked, axis=1, keepdims=True)
    i2 = jnp.argmax(masked, axis=1)[:, None]
    e2 = jnp.exp(v2 - v1)
    g1 = 1.0 / (1.0 + e2)
    g2 = e2 * g1
    combine = jnp.where(lane == i1, g1, 0.0) + jnp.where(lane == i2, g2, 0.0)
    combine_ref[...] = combine

    ex = jnp.exp(logits - v1)
    probs = ex / jnp.sum(ex, axis=1, keepdims=True)
    probs_part_ref[0, 0, :] = jnp.sum(probs, axis=0)
    comb_part_ref[0, 0, :] = jnp.sum(combine, axis=0)

    code_bf_ref[...] = code_ref[...].astype(jnp.bfloat16)


def _moe_kernel(temp_ref, fi_ref, code_bf_ref, comb_ref,
                w1_ref, w2_ref,
                cmw_ref, fw1_ref, fw2_ref, Fw1_ref, Fw2_ref,
                imp_ref, out_ref, cmw_bf_ref, fw1_bf_ref, fw2_bf_ref,
                Fw1_bf_ref, Fw2_bf_ref):
    bf = jnp.bfloat16
    e = pl.program_id(0)

    t = jnp.clip(temp_ref[0, 0], 0.1, 5.0)
    fi = fi_ref[0] / t
    fm = jnp.max(fi, axis=1, keepdims=True)
    fe = jnp.exp(fi - fm)
    imp = fe / jnp.sum(fe, axis=1, keepdims=True)
    imp_ref[0] = imp
    w1s = (w1_ref[0] * imp[0][:, None]).astype(bf)

    x = code_bf_ref[...]
    h1 = jnp.maximum(
        jnp.dot(x, w1s, preferred_element_type=jnp.float32),
        0.0).astype(bf)
    y = jnp.dot(h1, w2_ref[0].astype(bf),
                preferred_element_type=jnp.float32)
    comb = comb_ref[...]
    lane = jax.lax.broadcasted_iota(jnp.int32, comb.shape, 1)
    col = jnp.sum(jnp.where(lane == e, comb, 0.0), axis=1, keepdims=True)

    @pl.when(e == 0)
    def _():
        out_ref[...] = col * y

    @pl.when(e != 0)
    def _():
        out_ref[...] += col * y

    cmw_bf_ref[...] = cmw_ref[...].astype(bf)
    fw1_bf_ref[...] = fw1_ref[...].astype(bf)
    fw2_bf_ref[...] = fw2_ref[...].astype(bf)
    Fw1_bf_ref[...] = Fw1_ref[...].astype(bf)
    Fw2_bf_ref[...] = Fw2_ref[...].astype(bf)


def _softplus(x):
    return jnp.maximum(x, 0.0) + jnp.log1p(jnp.exp(-jnp.abs(x)))


def _head_kernel(moe_ref, code_bf_ref, u_ref, cmwc_ref, cmwu_ref,
                 fw1_ref, fw2_ref,
                 Fw1_ref, Fw2_ref,
                 probs_part_ref, comb_part_ref, imp_ref,
                 y_ref, losses_ref, spars_acc):
    bf = jnp.bfloat16
    i = pl.program_id(0)
    n = pl.num_programs(0)
    ub = u_ref[...].astype(bf)
    mask = jax.nn.sigmoid(
        jnp.dot(code_bf_ref[...], cmwc_ref[...],
                preferred_element_type=jnp.float32)
        + jnp.dot(ub, cmwu_ref[...], preferred_element_type=jnp.float32))
    f1 = jnp.maximum(
        jnp.dot(ub, fw1_ref[...], preferred_element_type=jnp.float32), 0.0)
    f_u = jnp.dot(f1.astype(bf), fw2_ref[...],
                  preferred_element_type=jnp.float32)
    out = moe_ref[...] * (1.0 - mask) + f_u * mask
    h = jnp.dot(out.astype(bf), Fw1_ref[...],
                preferred_element_type=jnp.float32)
    z = jnp.exp(jnp.minimum(h, 20.0))
    t2 = z * (z + 2.0)
    h = h * (t2 / (t2 + 2.0))
    y_ref[...] = jnp.tanh(
        jnp.dot(h.astype(bf), Fw2_ref[...],
                preferred_element_type=jnp.float32))

    part = jnp.broadcast_to(jnp.sum(jnp.abs(mask)), spars_acc.shape)

    @pl.when(i == 0)
    def _():
        spars_acc[...] = part

    @pl.when(i != 0)
    def _():
        spars_acc[...] += part

    @pl.when(i == n - 1)
    def _():
        T = moe_ref.shape[0] * n
        H = moe_ref.shape[1]
        E = probs_part_ref.shape[2]
        probs_mean = jnp.sum(probs_part_ref[...], axis=(0, 1)) / T
        load = jnp.sum(comb_part_ref[...], axis=(0, 1)) / T
        lb_loss = E * jnp.sum(probs_mean * load)
        impv = imp_ref[...]
        imp_loss = IMPORTANCE_REG * jnp.mean(jnp.sum(impv * impv, axis=-1))
        aux = lb_loss + imp_loss
        spars = jnp.max(spars_acc[...]) / (T * H)
        lane1 = jax.lax.broadcasted_iota(jnp.int32, losses_ref.shape, 1)
        v = jnp.where(lane1 == 0, aux + spars,
                      jnp.where(lane1 == 1, aux, spars))
        losses_ref[...] = v


def kernel(h_modulated, code_emb, u, gate_w, gate_b, feature_importance,
           importance_temperature, expert_w1, expert_b1, expert_w2, expert_b2,
           conf_mask_w, conf_mask_b, f_conf_w1, f_conf_b1, f_conf_w2,
           f_conf_b2, final_w1, final_b1, final_w2, final_b2):
    T, D = h_modulated.shape
    C = code_emb.shape[1]
    U = u.shape[1]
    E = gate_w.shape[1]
    H = expert_b1.shape[1]
    G = T // _TT
    bf = jnp.bfloat16

    temp = importance_temperature.reshape(1, 1)
    fi3 = feature_importance.reshape(E, 1, C)

    combine, probs_part, comb_part, code_bf = pl.pallas_call(
        _gate_kernel,
        grid=(G,),
        in_specs=[
            pl.BlockSpec((_TT, D), lambda i: (i, 0)),
            pl.BlockSpec((D, E), lambda i: (0, 0)),
            pl.BlockSpec((E,), lambda i: (0,)),
            pl.BlockSpec((_TT, C), lambda i: (i, 0)),
        ],
        out_specs=[
            pl.BlockSpec((_TT, E), lambda i: (i, 0)),
            pl.BlockSpec((1, 1, E), lambda i: (i, 0, 0)),
            pl.BlockSpec((1, 1, E), lambda i: (i, 0, 0)),
            pl.BlockSpec((_TT, C), lambda i: (i, 0)),
        ],
        out_shape=[
            jax.ShapeDtypeStruct((T, E), jnp.float32),
            jax.ShapeDtypeStruct((G, 1, E), jnp.float32),
            jax.ShapeDtypeStruct((G, 1, E), jnp.float32),
            jax.ShapeDtypeStruct((T, C), bf),
        ],
    )(h_modulated, gate_w, gate_b, code_emb)

    CMR = (C + U) // E
    FW1R = U // E
    FW2R = H // E
    FR1 = H // E
    FR2 = 2 * H // E

    (imp, moe_out, cmw_bf, fw1_bf, fw2_bf, Fw1_bf, Fw2_bf) = pl.pallas_call(
        _moe_kernel,
        grid=(E,),
        in_specs=[
            pl.BlockSpec(memory_space=pltpu.SMEM),
            pl.BlockSpec((1, 1, C), lambda e: (e, 0, 0)),
            pl.BlockSpec((T, C), lambda e: (0, 0)),
            pl.BlockSpec((T, E), lambda e: (0, 0)),
            pl.BlockSpec((1, C, H), lambda e: (e, 0, 0)),
            pl.BlockSpec((1, H, H), lambda e: (e, 0, 0)),
            pl.BlockSpec((CMR, H), lambda e: (e, 0)),
            pl.BlockSpec((FW1R, H), lambda e: (e, 0)),
            pl.BlockSpec((FW2R, H), lambda e: (e, 0)),
            pl.BlockSpec((FR1, 2 * H), lambda e: (e, 0)),
            pl.BlockSpec((FR2, H), lambda e: (e, 0)),
        ],
        out_specs=[
            pl.BlockSpec((1, 1, C), lambda e: (e, 0, 0)),
            pl.BlockSpec((T, H), lambda e: (0, 0)),
            pl.BlockSpec((CMR, H), lambda e: (e, 0)),
            pl.BlockSpec((FW1R, H), lambda e: (e, 0)),
            pl.BlockSpec((FW2R, H), lambda e: (e, 0)),
            pl.BlockSpec((FR1, 2 * H), lambda e: (e, 0)),
            pl.BlockSpec((FR2, H), lambda e: (e, 0)),
        ],
        out_shape=[
            jax.ShapeDtypeStruct((E, 1, C), jnp.float32),
            jax.ShapeDtypeStruct((T, H), jnp.float32),
            jax.ShapeDtypeStruct((C + U, H), bf),
            jax.ShapeDtypeStruct((U, H), bf),
            jax.ShapeDtypeStruct((H, H), bf),
            jax.ShapeDtypeStruct((H, 2 * H), bf),
            jax.ShapeDtypeStruct((2 * H, H), bf),
        ],
    )(temp, fi3, code_bf, combine, expert_w1, expert_w2,
      conf_mask_w, f_conf_w1, f_conf_w2, final_w1, final_w2)

    y, losses = pl.pallas_call(
        _head_kernel,
        grid=(T // _TH,),
        in_specs=[
            pl.BlockSpec((_TH, H), lambda i: (i, 0)),
            pl.BlockSpec((_TH, C), lambda i: (i, 0)),
            pl.BlockSpec((_TH, U), lambda i: (i, 0)),
            pl.BlockSpec((C, H), lambda i: (0, 0)),
            pl.BlockSpec((U, H), lambda i: (C // U, 0)),
            pl.BlockSpec((U, H), lambda i: (0, 0)),
            pl.BlockSpec((H, H), lambda i: (0, 0)),
            pl.BlockSpec((H, 2 * H), lambda i: (0, 0)),
            pl.BlockSpec((2 * H, H), lambda i: (0, 0)),
            pl.BlockSpec((G, 1, E), lambda i: (0, 0, 0)),
            pl.BlockSpec((G, 1, E), lambda i: (0, 0, 0)),
            pl.BlockSpec((E, 1, C), lambda i: (0, 0, 0)),
        ],
        out_specs=[
            pl.BlockSpec((_TH, H), lambda i: (i, 0)),
            pl.BlockSpec((1, 128), lambda i: (0, 0)),
        ],
        out_shape=[
            jax.ShapeDtypeStruct((T, H), jnp.float32),
            jax.ShapeDtypeStruct((1, 128), jnp.float32),
        ],
        scratch_shapes=[pltpu.VMEM((1, 128), jnp.float32)],
    )(moe_out, code_bf, u, cmw_bf, cmw_bf,
      fw1_bf, fw2_bf,
      Fw1_bf, Fw2_bf,
      probs_part, comb_part, imp)

    total_loss = losses[0, 0]
    aux_loss = losses[0, 1]
    sparsity_loss = losses[0, 2]
    return (y, total_loss, aux_loss, sparsity_loss)


# speedup vs baseline: 1.0330x; 1.0113x over previous
import jax
import jax.numpy as jnp
from jax.experimental import pallas as pl
from jax.experimental.pallas import tpu as pltpu

TOP_K = 2
IMPORTANCE_REG = 0.01

_TT = 512
_TH = 512


def _gate_kernel(hm_ref, gw_ref, gb_ref, code_ref,
                 combine_ref, probs_part_ref, comb_part_ref, code_bf_ref):
    logits = jnp.dot(hm_ref[...], gw_ref[...],
                     preferred_element_type=jnp.float32) + gb_ref[...][None, :]
    lane = jax.lax.broadcasted_iota(jnp.int32, logits.shape, 1)
    v1 = jnp.max(logits, axis=1, keepdims=True)
    i1 = jnp.argmax(logits, axis=1)[:, None]
    masked = jnp.where(lane == i1, -jnp.inf, logits)
    v2 = jnp.max(masked, axis=1, keepdims=True)
    i2 = jnp.argmax(masked, axis=1)[:, None]
    e2 = jnp.exp(v2 - v1)
    g1 = 1.0 / (1.0 + e2)
    g2 = e2 * g1
    combine = jnp.where(lane == i1, g1, 0.0) + jnp.where(lane == i2, g2, 0.0)
    combine_ref[...] = combine

    ex = jnp.exp(logits - v1)
    probs = ex / jnp.sum(ex, axis=1, keepdims=True)
    probs_part_ref[0, 0, :] = jnp.sum(probs, axis=0)
    comb_part_ref[0, 0, :] = jnp.sum(combine, axis=0)

    code_bf_ref[...] = code_ref[...].astype(jnp.bfloat16)


def _moe_kernel(temp_ref, fi_ref, code_bf_ref, comb_ref,
                w1_ref, w2_ref,
                cmw_ref, fw1_ref, fw2_ref, Fw1_ref, Fw2_ref,
                imp_ref, out_ref, cmw_bf_ref, fw1_bf_ref, fw2_bf_ref,
                Fw1_bf_ref, Fw2_bf_ref):
    bf = jnp.bfloat16
    e = pl.program_id(0)

    t = jnp.clip(temp_ref[0, 0], 0.1, 5.0)
    fi = fi_ref[0] / t
    fm = jnp.max(fi, axis=1, keepdims=True)
    fe = jnp.exp(fi - fm)
    imp = fe / jnp.sum(fe, axis=1, keepdims=True)
    imp_ref[0] = imp
    w1s = (w1_ref[0] * imp[0][:, None]).astype(bf)

    x = code_bf_ref[...]
    h1 = jnp.maximum(
        jnp.dot(x, w1s, preferred_element_type=jnp.float32),
        0.0).astype(bf)
    y = jnp.dot(h1, w2_ref[0].astype(bf),
                preferred_element_type=jnp.float32)
    comb = comb_ref[...]
    lane = jax.lax.broadcasted_iota(jnp.int32, comb.shape, 1)
    col = jnp.sum(jnp.where(lane == e, comb, 0.0), axis=1, keepdims=True)

    @pl.when(e == 0)
    def _():
        out_ref[...] = col * y

    @pl.when(e != 0)
    def _():
        out_ref[...] += col * y

    cmw_bf_ref[...] = cmw_ref[...].astype(bf)
    fw1_bf_ref[...] = fw1_ref[...].astype(bf)
    fw2_bf_ref[...] = fw2_ref[...].astype(bf)
    Fw1_bf_ref[...] = Fw1_ref[...].astype(bf)
    Fw2_bf_ref[...] = Fw2_ref[...].astype(bf)


def _softplus(x):
    return jnp.maximum(x, 0.0) + jnp.log1p(jnp.exp(-jnp.abs(x)))


def _head_kernel(moe_ref, code_bf_ref, u_ref, cmwc_ref, cmwu_ref,
                 fw1_ref, fw2_ref,
                 Fw1_ref, Fw2_ref,
                 probs_part_ref, comb_part_ref, imp_ref,
                 y_ref, losses_ref, spars_acc):
    bf = jnp.bfloat16
    i = pl.program_id(0)
    n = pl.num_programs(0)
    ub = u_ref[...].astype(bf)
    mask = jax.nn.sigmoid(
        jnp.dot(code_bf_ref[...], cmwc_ref[...],
                preferred_element_type=jnp.float32)
        + jnp.dot(ub, cmwu_ref[...], preferred_element_type=jnp.float32))
    f1 = jnp.maximum(
        jnp.dot(ub, fw1_ref[...], preferred_element_type=jnp.float32), 0.0)
    f_u = jnp.dot(f1.astype(bf), fw2_ref[...],
                  preferred_element_type=jnp.float32)
    out = moe_ref[...] * (1.0 - mask) + f_u * mask
    h = jnp.dot(out.astype(bf), Fw1_ref[...],
                preferred_element_type=jnp.float32)
    z = jnp.exp(jnp.minimum(h, 20.0))
    t2 = z * (z + 2.0)
    h = h * (t2 / (t2 + 2.0))
    y_ref[...] = jnp.tanh(
        jnp.dot(h.astype(bf), Fw2_ref[...],
                preferred_element_type=jnp.float32))

    part = jnp.broadcast_to(jnp.sum(jnp.abs(mask)), spars_acc.shape)

    @pl.when(i == 0)
    def _():
        spars_acc[...] = part

    @pl.when(i != 0)
    def _():
        spars_acc[...] += part

    @pl.when(i == n - 1)
    def _():
        T = moe_ref.shape[0] * n
        H = moe_ref.shape[1]
        E = probs_part_ref.shape[2]
        probs_mean = jnp.sum(probs_part_ref[...], axis=(0, 1)) / T
        load = jnp.sum(comb_part_ref[...], axis=(0, 1)) / T
        lb_loss = E * jnp.sum(probs_mean * load)
        impv = imp_ref[...]
        imp_loss = IMPORTANCE_REG * jnp.mean(jnp.sum(impv * impv, axis=-1))
        aux = lb_loss + imp_loss
        spars = jnp.max(spars_acc[...]) / (T * H)
        lane1 = jax.lax.broadcasted_iota(jnp.int32, losses_ref.shape, 1)
        v = jnp.where(lane1 == 0, aux + spars,
                      jnp.where(lane1 == 1, aux, spars))
        losses_ref[...] = v


def kernel(h_modulated, code_emb, u, gate_w, gate_b, feature_importance,
           importance_temperature, expert_w1, expert_b1, expert_w2, expert_b2,
           conf_mask_w, conf_mask_b, f_conf_w1, f_conf_b1, f_conf_w2,
           f_conf_b2, final_w1, final_b1, final_w2, final_b2):
    T, D = h_modulated.shape
    C = code_emb.shape[1]
    U = u.shape[1]
    E = gate_w.shape[1]
    H = expert_b1.shape[1]
    G = T // _TT
    bf = jnp.bfloat16

    temp = importance_temperature.reshape(1, 1)
    fi3 = feature_importance.reshape(E, 1, C)

    combine, probs_part, comb_part, code_bf = pl.pallas_call(
        _gate_kernel,
        grid=(G,),
        in_specs=[
            pl.BlockSpec((_TT, D), lambda i: (i, 0)),
            pl.BlockSpec((D, E), lambda i: (0, 0)),
            pl.BlockSpec((E,), lambda i: (0,)),
            pl.BlockSpec((_TT, C), lambda i: (i, 0)),
        ],
        out_specs=[
            pl.BlockSpec((_TT, E), lambda i: (i, 0)),
            pl.BlockSpec((1, 1, E), lambda i: (i, 0, 0)),
            pl.BlockSpec((1, 1, E), lambda i: (i, 0, 0)),
            pl.BlockSpec((_TT, C), lambda i: (i, 0)),
        ],
        out_shape=[
            jax.ShapeDtypeStruct((T, E), jnp.float32),
            jax.ShapeDtypeStruct((G, 1, E), jnp.float32),
            jax.ShapeDtypeStruct((G, 1, E), jnp.float32),
            jax.ShapeDtypeStruct((T, C), bf),
        ],
    )(h_modulated, gate_w, gate_b, code_emb)

    CMR = (C + U) // E
    FW1R = U // E
    FW2R = H // E
    FR1 = H // E
    FR2 = 2 * H // E

    (imp, moe_out, cmw_bf, fw1_bf, fw2_bf, Fw1_bf, Fw2_bf) = pl.pallas_call(
        _moe_kernel,
        grid=(E,),
        in_specs=[
            pl.BlockSpec(memory_space=pltpu.SMEM),
            pl.BlockSpec((1, 1, C), lambda e: (e, 0, 0)),
            pl.BlockSpec((T, C), lambda e: (0, 0)),
            pl.BlockSpec((T, E), lambda e: (0, 0)),
            pl.BlockSpec((1, C, H), lambda e: (e, 0, 0)),
            pl.BlockSpec((1, H, H), lambda e: (e, 0, 0)),
            pl.BlockSpec((CMR, H), lambda e: (e, 0)),
            pl.BlockSpec((FW1R, H), lambda e: (e, 0)),
            pl.BlockSpec((FW2R, H), lambda e: (e, 0)),
            pl.BlockSpec((FR1, 2 * H), lambda e: (e, 0)),
            pl.BlockSpec((FR2, H), lambda e: (e, 0)),
        ],
        out_specs=[
            pl.BlockSpec((1, 1, C), lambda e: (e, 0, 0)),
            pl.BlockSpec((T, H), lambda e: (0, 0)),
            pl.BlockSpec((CMR, H), lambda e: (e, 0)),
            pl.BlockSpec((FW1R, H), lambda e: (e, 0)),
            pl.BlockSpec((FW2R, H), lambda e: (e, 0)),
            pl.BlockSpec((FR1, 2 * H), lambda e: (e, 0)),
            pl.BlockSpec((FR2, H), lambda e: (e, 0)),
        ],
        out_shape=[
            jax.ShapeDtypeStruct((E, 1, C), jnp.float32),
            jax.ShapeDtypeStruct((T, H), jnp.float32),
            jax.ShapeDtypeStruct((C + U, H), bf),
            jax.ShapeDtypeStruct((U, H), bf),
            jax.ShapeDtypeStruct((H, H), bf),
            jax.ShapeDtypeStruct((H, 2 * H), bf),
            jax.ShapeDtypeStruct((2 * H, H), bf),
        ],
    )(temp, fi3, code_bf, combine, expert_w1, expert_w2,
      conf_mask_w, f_conf_w1, f_conf_w2, final_w1, final_w2)

    y, losses = pl.pallas_call(
        _head_kernel,
        grid=(T // _TH,),
        in_specs=[
            pl.BlockSpec((_TH, H), lambda i: (i, 0)),
            pl.BlockSpec((_TH, C), lambda i: (i, 0)),
            pl.BlockSpec((_TH, U), lambda i: (i, 0)),
            pl.BlockSpec((C, H), lambda i: (0, 0)),
            pl.BlockSpec((U, H), lambda i: (C // U, 0)),
            pl.BlockSpec((U, H), lambda i: (0, 0)),
            pl.BlockSpec((H, H), lambda i: (0, 0)),
            pl.BlockSpec((H, 2 * H), lambda i: (0, 0)),
            pl.BlockSpec((2 * H, H), lambda i: (0, 0)),
            pl.BlockSpec((G, 1, E), lambda i: (0, 0, 0)),
            pl.BlockSpec((G, 1, E), lambda i: (0, 0, 0)),
            pl.BlockSpec((E, 1, C), lambda i: (0, 0, 0)),
        ],
        out_specs=[
            pl.BlockSpec((_TH, H), lambda i: (i, 0)),
            pl.BlockSpec((1, 128), lambda i: (0, 0)),
        ],
        out_shape=[
            jax.ShapeDtypeStruct((T, H), jnp.float32),
            jax.ShapeDtypeStruct((1, 128), jnp.float32),
        ],
        scratch_shapes=[pltpu.VMEM((1, 128), jnp.float32)],
    )(moe_out, code_bf, u, cmw_bf, cmw_bf,
      fw1_bf, fw2_bf,
      Fw1_bf, Fw2_bf,
      probs_part, comb_part, imp)

    total_loss = losses[0, 0]
    aux_loss = losses[0, 1]
    sparsity_loss = losses[0, 2]
    return (y, total_loss, aux_loss, sparsity_loss)


# speedup vs baseline: 1.0404x; 1.0071x over previous
import jax
import jax.numpy as jnp
from jax.experimental import pallas as pl
from jax.experimental.pallas import tpu as pltpu

TOP_K = 2
IMPORTANCE_REG = 0.01

_TT = 1024
_TH = 512


def _gate_kernel(hm_ref, gw_ref, gb_ref, code_ref,
                 combine_ref, probs_part_ref, comb_part_ref, code_bf_ref):
    logits = jnp.dot(hm_ref[...], gw_ref[...],
                     preferred_element_type=jnp.float32) + gb_ref[...][None, :]
    lane = jax.lax.broadcasted_iota(jnp.int32, logits.shape, 1)
    v1 = jnp.max(logits, axis=1, keepdims=True)
    i1 = jnp.argmax(logits, axis=1)[:, None]
    masked = jnp.where(lane == i1, -jnp.inf, logits)
    v2 = jnp.max(masked, axis=1, keepdims=True)
    i2 = jnp.argmax(masked, axis=1)[:, None]
    e2 = jnp.exp(v2 - v1)
    g1 = 1.0 / (1.0 + e2)
    g2 = e2 * g1
    combine = jnp.where(lane == i1, g1, 0.0) + jnp.where(lane == i2, g2, 0.0)
    combine_ref[...] = combine

    ex = jnp.exp(logits - v1)
    probs = ex / jnp.sum(ex, axis=1, keepdims=True)
    probs_part_ref[0, 0, :] = jnp.sum(probs, axis=0)
    comb_part_ref[0, 0, :] = jnp.sum(combine, axis=0)

    code_bf_ref[...] = code_ref[...].astype(jnp.bfloat16)


def _moe_kernel(temp_ref, fi_ref, code_bf_ref, comb_ref,
                w1_ref, w2_ref,
                cmw_ref, fw1_ref, fw2_ref, Fw1_ref, Fw2_ref,
                imp_ref, out_ref, cmw_bf_ref, fw1_bf_ref, fw2_bf_ref,
                Fw1_bf_ref, Fw2_bf_ref):
    bf = jnp.bfloat16
    e = pl.program_id(0)

    t = jnp.clip(temp_ref[0, 0], 0.1, 5.0)
    fi = fi_ref[0] / t
    fm = jnp.max(fi, axis=1, keepdims=True)
    fe = jnp.exp(fi - fm)
    imp = fe / jnp.sum(fe, axis=1, keepdims=True)
    imp_ref[0] = imp
    w1s = (w1_ref[0] * imp[0][:, None]).astype(bf)

    x = code_bf_ref[...]
    h1 = jnp.maximum(
        jnp.dot(x, w1s, preferred_element_type=jnp.float32),
        0.0).astype(bf)
    y = jnp.dot(h1, w2_ref[0].astype(bf),
                preferred_element_type=jnp.float32)
    comb = comb_ref[...]
    lane = jax.lax.broadcasted_iota(jnp.int32, comb.shape, 1)
    col = jnp.sum(jnp.where(lane == e, comb, 0.0), axis=1, keepdims=True)

    @pl.when(e == 0)
    def _():
        out_ref[...] = col * y

    @pl.when(e != 0)
    def _():
        out_ref[...] += col * y

    cmw_bf_ref[...] = cmw_ref[...].astype(bf)
    fw1_bf_ref[...] = fw1_ref[...].astype(bf)
    fw2_bf_ref[...] = fw2_ref[...].astype(bf)
    Fw1_bf_ref[...] = Fw1_ref[...].astype(bf)
    Fw2_bf_ref[...] = Fw2_ref[...].astype(bf)


def _softplus(x):
    return jnp.maximum(x, 0.0) + jnp.log1p(jnp.exp(-jnp.abs(x)))


def _head_kernel(moe_ref, code_bf_ref, u_ref, cmwc_ref, cmwu_ref,
                 fw1_ref, fw2_ref,
                 Fw1_ref, Fw2_ref,
                 probs_part_ref, comb_part_ref, imp_ref,
                 y_ref, losses_ref, spars_acc):
    bf = jnp.bfloat16
    i = pl.program_id(0)
    n = pl.num_programs(0)
    ub = u_ref[...].astype(bf)
    mask = jax.nn.sigmoid(
        jnp.dot(code_bf_ref[...], cmwc_ref[...],
                preferred_element_type=jnp.float32)
        + jnp.dot(ub, cmwu_ref[...], preferred_element_type=jnp.float32))
    f1 = jnp.maximum(
        jnp.dot(ub, fw1_ref[...], preferred_element_type=jnp.float32), 0.0)
    f_u = jnp.dot(f1.astype(bf), fw2_ref[...],
                  preferred_element_type=jnp.float32)
    out = moe_ref[...] * (1.0 - mask) + f_u * mask
    h = jnp.dot(out.astype(bf), Fw1_ref[...],
                preferred_element_type=jnp.float32)
    z = jnp.exp(jnp.minimum(h, 20.0))
    t2 = z * (z + 2.0)
    h = h * (t2 / (t2 + 2.0))
    y_ref[...] = jnp.tanh(
        jnp.dot(h.astype(bf), Fw2_ref[...],
                preferred_element_type=jnp.float32))

    part = jnp.broadcast_to(jnp.sum(jnp.abs(mask)), spars_acc.shape)

    @pl.when(i == 0)
    def _():
        spars_acc[...] = part

    @pl.when(i != 0)
    def _():
        spars_acc[...] += part

    @pl.when(i == n - 1)
    def _():
        T = moe_ref.shape[0] * n
        H = moe_ref.shape[1]
        E = probs_part_ref.shape[2]
        probs_mean = jnp.sum(probs_part_ref[...], axis=(0, 1)) / T
        load = jnp.sum(comb_part_ref[...], axis=(0, 1)) / T
        lb_loss = E * jnp.sum(probs_mean * load)
        impv = imp_ref[...]
        imp_loss = IMPORTANCE_REG * jnp.mean(jnp.sum(impv * impv, axis=-1))
        aux = lb_loss + imp_loss
        spars = jnp.max(spars_acc[...]) / (T * H)
        lane1 = jax.lax.broadcasted_iota(jnp.int32, losses_ref.shape, 1)
        v = jnp.where(lane1 == 0, aux + spars,
                      jnp.where(lane1 == 1, aux, spars))
        losses_ref[...] = v


def kernel(h_modulated, code_emb, u, gate_w, gate_b, feature_importance,
           importance_temperature, expert_w1, expert_b1, expert_w2, expert_b2,
           conf_mask_w, conf_mask_b, f_conf_w1, f_conf_b1, f_conf_w2,
           f_conf_b2, final_w1, final_b1, final_w2, final_b2):
    T, D = h_modulated.shape
    C = code_emb.shape[1]
    U = u.shape[1]
    E = gate_w.shape[1]
    H = expert_b1.shape[1]
    G = T // _TT
    bf = jnp.bfloat16

    temp = importance_temperature.reshape(1, 1)
    fi3 = feature_importance.reshape(E, 1, C)

    combine, probs_part, comb_part, code_bf = pl.pallas_call(
        _gate_kernel,
        grid=(G,),
        in_specs=[
            pl.BlockSpec((_TT, D), lambda i: (i, 0)),
            pl.BlockSpec((D, E), lambda i: (0, 0)),
            pl.BlockSpec((E,), lambda i: (0,)),
            pl.BlockSpec((_TT, C), lambda i: (i, 0)),
        ],
        out_specs=[
            pl.BlockSpec((_TT, E), lambda i: (i, 0)),
            pl.BlockSpec((1, 1, E), lambda i: (i, 0, 0)),
            pl.BlockSpec((1, 1, E), lambda i: (i, 0, 0)),
            pl.BlockSpec((_TT, C), lambda i: (i, 0)),
        ],
        out_shape=[
            jax.ShapeDtypeStruct((T, E), jnp.float32),
            jax.ShapeDtypeStruct((G, 1, E), jnp.float32),
            jax.ShapeDtypeStruct((G, 1, E), jnp.float32),
            jax.ShapeDtypeStruct((T, C), bf),
        ],
    )(h_modulated, gate_w, gate_b, code_emb)

    CMR = (C + U) // E
    FW1R = U // E
    FW2R = H // E
    FR1 = H // E
    FR2 = 2 * H // E

    (imp, moe_out, cmw_bf, fw1_bf, fw2_bf, Fw1_bf, Fw2_bf) = pl.pallas_call(
        _moe_kernel,
        grid=(E,),
        in_specs=[
            pl.BlockSpec(memory_space=pltpu.SMEM),
            pl.BlockSpec((1, 1, C), lambda e: (e, 0, 0)),
            pl.BlockSpec((T, C), lambda e: (0, 0)),
            pl.BlockSpec((T, E), lambda e: (0, 0)),
            pl.BlockSpec((1, C, H), lambda e: (e, 0, 0)),
            pl.BlockSpec((1, H, H), lambda e: (e, 0, 0)),
            pl.BlockSpec((CMR, H), lambda e: (e, 0)),
            pl.BlockSpec((FW1R, H), lambda e: (e, 0)),
            pl.BlockSpec((FW2R, H), lambda e: (e, 0)),
            pl.BlockSpec((FR1, 2 * H), lambda e: (e, 0)),
            pl.BlockSpec((FR2, H), lambda e: (e, 0)),
        ],
        out_specs=[
            pl.BlockSpec((1, 1, C), lambda e: (e, 0, 0)),
            pl.BlockSpec((T, H), lambda e: (0, 0)),
            pl.BlockSpec((CMR, H), lambda e: (e, 0)),
            pl.BlockSpec((FW1R, H), lambda e: (e, 0)),
            pl.BlockSpec((FW2R, H), lambda e: (e, 0)),
            pl.BlockSpec((FR1, 2 * H), lambda e: (e, 0)),
            pl.BlockSpec((FR2, H), lambda e: (e, 0)),
        ],
        out_shape=[
            jax.ShapeDtypeStruct((E, 1, C), jnp.float32),
            jax.ShapeDtypeStruct((T, H), jnp.float32),
            jax.ShapeDtypeStruct((C + U, H), bf),
            jax.ShapeDtypeStruct((U, H), bf),
            jax.ShapeDtypeStruct((H, H), bf),
            jax.ShapeDtypeStruct((H, 2 * H), bf),
            jax.ShapeDtypeStruct((2 * H, H), bf),
        ],
    )(temp, fi3, code_bf, combine, expert_w1, expert_w2,
      conf_mask_w, f_conf_w1, f_conf_w2, final_w1, final_w2)

    y, losses = pl.pallas_call(
        _head_kernel,
        grid=(T // _TH,),
        in_specs=[
            pl.BlockSpec((_TH, H), lambda i: (i, 0)),
            pl.BlockSpec((_TH, C), lambda i: (i, 0)),
            pl.BlockSpec((_TH, U), lambda i: (i, 0)),
            pl.BlockSpec((C, H), lambda i: (0, 0)),
            pl.BlockSpec((U, H), lambda i: (C // U, 0)),
            pl.BlockSpec((U, H), lambda i: (0, 0)),
            pl.BlockSpec((H, H), lambda i: (0, 0)),
            pl.BlockSpec((H, 2 * H), lambda i: (0, 0)),
            pl.BlockSpec((2 * H, H), lambda i: (0, 0)),
            pl.BlockSpec((G, 1, E), lambda i: (0, 0, 0)),
            pl.BlockSpec((G, 1, E), lambda i: (0, 0, 0)),
            pl.BlockSpec((E, 1, C), lambda i: (0, 0, 0)),
        ],
        out_specs=[
            pl.BlockSpec((_TH, H), lambda i: (i, 0)),
            pl.BlockSpec((1, 128), lambda i: (0, 0)),
        ],
        out_shape=[
            jax.ShapeDtypeStruct((T, H), jnp.float32),
            jax.ShapeDtypeStruct((1, 128), jnp.float32),
        ],
        scratch_shapes=[pltpu.VMEM((1, 128), jnp.float32)],
    )(moe_out, code_bf, u, cmw_bf, cmw_bf,
      fw1_bf, fw2_bf,
      Fw1_bf, Fw2_bf,
      probs_part, comb_part, imp)

    total_loss = losses[0, 0]
    aux_loss = losses[0, 1]
    sparsity_loss = losses[0, 2]
    return (y, total_loss, aux_loss, sparsity_loss)
